# baseline (speedup 1.0000x reference)
import functools
from itertools import combinations

import numpy as np
import jax
import jax.numpy as jnp
from jax import lax
from jax.experimental import pallas as pl
from jax.experimental.pallas import tpu as pltpu
from jax.experimental.pallas import tpu_sc as plsc

_SIZE_IN = 14
_HOTNESS = 3
_BATCH = 4096
_IN_DIM = 2 ** _SIZE_IN
_NCOL = 364
_LN = 16

_NW = 32
_ROWS_PER_W = _BATCH // _NW
_RB = 16
_NBLK = _ROWS_PER_W // _RB
_HALF = 96
_M = _RB * 2
_NG = 23
_CHUNKS_PER_ROW = _IN_DIM // _LN

def _build_constants():
    idx = np.array(
        [sum(2 ** i for i in ones) for ones in combinations(range(_SIZE_IN), _HOTNESS)],
        dtype=np.int32,
    )
    chunks = np.unique(idx >> 4)
    nch = len(chunks)
    chunks_pad = np.zeros(2 * _HALF, np.int32)
    chunks_pad[:nch] = chunks
    pos = {int(c): p for p, c in enumerate(chunks)}

    gstart = [g * _LN for g in range(_NCOL // _LN)] + [_NCOL - _LN]
    selr = np.zeros(_NG * _LN, np.int32)
    sell = np.zeros(_NG * _LN, np.int32)
    for g, s in enumerate(gstart):
        for t in range(_LN):
            p = pos[int(idx[s + t]) >> 4]
            selr[g * _LN + t] = (p // _HALF) * _HALF + p % _HALF
            sell[g * _LN + t] = idx[s + t] & (_LN - 1)

    pat = np.zeros((_M, _HALF), np.int32)
    for r in range(_RB):
        for h in range(2):
            pat[2 * r + h, :] = r * _CHUNKS_PER_ROW + chunks_pad[h * _HALF:(h + 1) * _HALF]
    return gstart, pat, selr, sell


_GSTART, _PAT, _SELR, _SELL = _build_constants()

_mesh = plsc.VectorSubcoreMesh(core_axis_name="c", subcore_axis_name="s")


@functools.partial(
    pl.kernel,
    out_type=jax.ShapeDtypeStruct((_BATCH, _NCOL), jnp.float32),
    mesh=_mesh,
    compiler_params=pltpu.CompilerParams(needs_layout_passes=False,
                                         use_tc_tiling_on_sc=False),
    scratch_types=[
        pltpu.VMEM((_M, _HALF), jnp.int32),
        pltpu.VMEM((_M * _HALF, _LN), jnp.float32),
        pltpu.VMEM((_RB, _NCOL), jnp.float32),
        pltpu.VMEM((_M, _HALF), jnp.int32),
        pltpu.VMEM((_NG * _LN,), jnp.int32),
        pltpu.VMEM((_NG * _LN,), jnp.int32),
        pltpu.SemaphoreType.DMA,
    ],
)
def _gather_affine(table, pat_hbm, selr_hbm, sell_hbm, out,
                   idx_v, st_v, out_v, pat_v, selr_v, sell_v, sem):
    wid = lax.axis_index("s") * 2 + lax.axis_index("c")
    pltpu.sync_copy(pat_hbm, pat_v)
    pltpu.sync_copy(selr_hbm, selr_v)
    pltpu.sync_copy(sell_hbm, sell_v)
    row0w = wid * _ROWS_PER_W

    def blk_body(b, carry):
        row0 = row0w + b * _RB
        off = row0 * _CHUNKS_PER_ROW

        def idx_body(m, c):
            for kk in range(_HALF // _LN):
                sl = pl.ds(kk * _LN, _LN)
                idx_v[m, sl] = pat_v[m, sl] + off
            return c

        lax.fori_loop(0, _M, idx_body, 0)

        copies = [
            pltpu.async_copy(table.at[idx_v.at[m]],
                             st_v.at[pl.ds(m * _HALF, _HALF)], sem)
            for m in range(_M)
        ]
        for c in copies:
            c.wait()

        def row_body(r, c):
            for g in range(_NG):
                sl = pl.ds(g * _LN, _LN)
                rr = selr_v[sl] + (2 * _HALF) * r
                ll = sell_v[sl]
                v = plsc.load_gather(st_v, [rr, ll])
                out_v[r, pl.ds(_GSTART[g], _LN)] = v * 12.0 - 6.0
            return c

        lax.fori_loop(0, _RB, row_body, 0)
        pltpu.sync_copy(out_v, out.at[pl.ds(row0, _RB)])
        return carry

    lax.fori_loop(0, _NBLK, blk_body, 0)


def kernel(input_var):
    table = input_var.reshape(_BATCH * _CHUNKS_PER_ROW, _LN)
    return _gather_affine(table, jnp.asarray(_PAT), jnp.asarray(_SELR),
                          jnp.asarray(_SELL))


# speedup vs baseline: 2.2981x; 2.2981x over previous
import functools
from itertools import combinations

import numpy as np
import jax
import jax.numpy as jnp
from jax import lax
from jax.experimental import pallas as pl
from jax.experimental.pallas import tpu as pltpu
from jax.experimental.pallas import tpu_sc as plsc

_SIZE_IN = 14
_HOTNESS = 3
_BATCH = 4096
_IN_DIM = 2 ** _SIZE_IN
_NCOL = 364
_LN = 16

_NW = 32
_RB = 8
_TROWS_PER_W = (_BATCH // _RB) // _NW
_NTILE = 64
_NG = 23


def _build_constants():
    idx = np.array(
        [sum(2 ** i for i in ones) for ones in combinations(range(_SIZE_IN), _HOTNESS)],
        dtype=np.int32,
    )
    tiles = np.unique(idx >> 7)
    assert len(tiles) == _NTILE
    pos = {int(t): p for p, t in enumerate(tiles)}

    gstart = [g * _LN for g in range(_NCOL // _LN)] + [_NCOL - _LN]
    selr = np.zeros(_NG * _LN, np.int32)
    sell = np.zeros(_NG * _LN, np.int32)
    for g, s in enumerate(gstart):
        for t in range(_LN):
            v = int(idx[s + t])
            selr[g * _LN + t] = pos[v >> 7] * _RB
            sell[g * _LN + t] = v & 127
    return gstart, [int(t) for t in tiles], selr, sell


_GSTART, _TILES, _SELR, _SELL = _build_constants()

_mesh = plsc.VectorSubcoreMesh(core_axis_name="c", subcore_axis_name="s")


@functools.partial(
    pl.kernel,
    out_type=jax.ShapeDtypeStruct((_BATCH * _NCOL,), jnp.float32),
    mesh=_mesh,
    compiler_params=pltpu.CompilerParams(needs_layout_passes=False,
                                         use_tc_tiling_on_sc=True),
    scratch_types=[
        pltpu.VMEM((_NTILE * _RB, 128), jnp.float32),
        pltpu.VMEM((_RB * _NCOL,), jnp.float32),
        pltpu.VMEM((_NG * _LN,), jnp.int32),
        pltpu.VMEM((_NG * _LN,), jnp.int32),
        pltpu.SemaphoreType.DMA,
    ],
)
def _gather_affine(table, selr_hbm, sell_hbm, out,
                   st_v, out_v, selr_v, sell_v, sem):
    wid = lax.axis_index("s") * 2 + lax.axis_index("c")
    pltpu.sync_copy(selr_hbm, selr_v)
    pltpu.sync_copy(sell_hbm, sell_v)
    tr0 = wid * _TROWS_PER_W

    def blk_body(b, carry):
        row0 = (tr0 + b) * _RB

        copies = [
            pltpu.async_copy(
                table.at[pl.ds(row0, _RB), pl.ds(_TILES[t] * 128, 128)],
                st_v.at[pl.ds(t * _RB, _RB)], sem)
            for t in range(_NTILE)
        ]
        for c in copies:
            c.wait()

        for g in range(_NG):
            sl = pl.ds(g * _LN, _LN)
            rr0 = selr_v[sl]
            ll = sell_v[sl]
            base = _GSTART[g]

            def row_body(r, c):
                v = plsc.load_gather(st_v, [rr0 + r, ll])
                out_v[pl.ds(r * _NCOL + base, _LN)] = v * 12.0 - 6.0
                return c

            lax.fori_loop(0, _RB, row_body, 0)

        pltpu.sync_copy(out_v, out.at[pl.ds(row0 * _NCOL, _RB * _NCOL)])
        return carry

    lax.fori_loop(0, _TROWS_PER_W, blk_body, 0)


def kernel(input_var):
    flat = _gather_affine(input_var, jnp.asarray(_SELR), jnp.asarray(_SELL))
    return flat.reshape(_BATCH, _NCOL)


# speedup vs baseline: 2.3528x; 1.0238x over previous
import functools
from itertools import combinations

import numpy as np
import jax
import jax.numpy as jnp
from jax import lax
from jax.experimental import pallas as pl
from jax.experimental.pallas import tpu as pltpu
from jax.experimental.pallas import tpu_sc as plsc

_SIZE_IN = 14
_HOTNESS = 3
_BATCH = 4096
_IN_DIM = 2 ** _SIZE_IN
_NCOL = 364
_LN = 16

_NW = 32
_RB = 8
_TROWS_PER_W = (_BATCH // _RB) // _NW
_NTILE = 64
_NG = 23


def _build_constants():
    idx = np.array(
        [sum(2 ** i for i in ones) for ones in combinations(range(_SIZE_IN), _HOTNESS)],
        dtype=np.int32,
    )
    tiles = np.unique(idx >> 7)
    assert len(tiles) == _NTILE
    pos = {int(t): p for p, t in enumerate(tiles)}

    spans = []
    s = prev = int(tiles[0])
    for t in tiles[1:]:
        t = int(t)
        if t == prev + 1:
            prev = t
        else:
            spans.append((s, pos[s], prev - s + 1))
            s = prev = t
    spans.append((s, pos[s], prev - s + 1))

    gstart = [g * _LN for g in range(_NCOL // _LN)] + [_NCOL - _LN]
    selc = np.zeros(_NG * _LN, np.int32)
    for g, st in enumerate(gstart):
        for t in range(_LN):
            v = int(idx[st + t])
            selc[g * _LN + t] = pos[v >> 7] * 128 + (v & 127)
    return gstart, spans, selc


_GSTART, _SPANS, _SELC = _build_constants()

_mesh = plsc.VectorSubcoreMesh(core_axis_name="c", subcore_axis_name="s")


@functools.partial(
    pl.kernel,
    out_type=jax.ShapeDtypeStruct((_BATCH * _NCOL,), jnp.float32),
    mesh=_mesh,
    compiler_params=pltpu.CompilerParams(needs_layout_passes=False,
                                         use_tc_tiling_on_sc=True),
    scratch_types=[
        pltpu.VMEM((_RB, _NTILE * 128), jnp.float32),
        pltpu.VMEM((_RB * _NCOL,), jnp.float32),
        pltpu.VMEM((_NG * _LN,), jnp.int32),
        pltpu.SemaphoreType.DMA,
    ],
)
def _gather_affine(table, selc_hbm, out, st_v, out_v, selc_v, sem):
    wid = lax.axis_index("s") * 2 + lax.axis_index("c")
    pltpu.sync_copy(selc_hbm, selc_v)
    tr0 = wid * _TROWS_PER_W

    def blk_body(b, carry):
        row0 = (tr0 + b) * _RB

        copies = [
            pltpu.async_copy(
                table.at[pl.ds(row0, _RB), pl.ds(ts * 128, n * 128)],
                st_v.at[:, pl.ds(ps * 128, n * 128)], sem)
            for (ts, ps, n) in _SPANS
        ]
        for c in copies:
            c.wait()

        for g in range(_NG):
            cc = selc_v[pl.ds(g * _LN, _LN)]
            base = _GSTART[g]
            for r in range(_RB):
                rr = jnp.full((_LN,), r, jnp.int32)
                v = plsc.load_gather(st_v, [rr, cc])
                out_v[pl.ds(r * _NCOL + base, _LN)] = v * 12.0 - 6.0

        pltpu.sync_copy(out_v, out.at[pl.ds(row0 * _NCOL, _RB * _NCOL)])
        return carry

    lax.fori_loop(0, _TROWS_PER_W, blk_body, 0)


def kernel(input_var):
    flat = _gather_affine(input_var, jnp.asarray(_SELC))
    return flat.reshape(_BATCH, _NCOL)


# speedup vs baseline: 2.5721x; 1.0932x over previous
import functools
from itertools import combinations

import numpy as np
import jax
import jax.numpy as jnp
from jax import lax
from jax.experimental import pallas as pl
from jax.experimental.pallas import tpu as pltpu
from jax.experimental.pallas import tpu_sc as plsc

_SIZE_IN = 14
_HOTNESS = 3
_BATCH = 4096
_IN_DIM = 2 ** _SIZE_IN
_NCOL = 364
_LN = 16

_NW = 32
_RB = 8
_TROWS_PER_W = (_BATCH // _RB) // _NW
_NTILE = 64
_NG = 23


def _build_constants():
    idx = np.array(
        [sum(2 ** i for i in ones) for ones in combinations(range(_SIZE_IN), _HOTNESS)],
        dtype=np.int32,
    )
    tiles = np.unique(idx >> 7)
    assert len(tiles) == _NTILE
    pos = {int(t): p for p, t in enumerate(tiles)}

    spans = []
    s = prev = int(tiles[0])
    for t in tiles[1:]:
        t = int(t)
        if t == prev + 1:
            prev = t
        else:
            spans.append((s, pos[s], prev - s + 1))
            s = prev = t
    spans.append((s, pos[s], prev - s + 1))

    gstart = [g * _LN for g in range(_NCOL // _LN)] + [_NCOL - _LN]
    selc = np.zeros(_NG * _LN, np.int32)
    for g, st in enumerate(gstart):
        for t in range(_LN):
            v = int(idx[st + t])
            selc[g * _LN + t] = pos[v >> 7] * 128 + (v & 127)
    return gstart, spans, selc


_GSTART, _SPANS, _SELC = _build_constants()

_mesh = plsc.VectorSubcoreMesh(core_axis_name="c", subcore_axis_name="s")


@functools.partial(
    pl.kernel,
    out_type=jax.ShapeDtypeStruct((_BATCH, _NCOL), jnp.float32),
    mesh=_mesh,
    compiler_params=pltpu.CompilerParams(needs_layout_passes=False,
                                         use_tc_tiling_on_sc=True),
    scratch_types=[
        pltpu.VMEM((_RB, _NTILE * 128), jnp.float32),
        pltpu.VMEM((_RB, _NCOL), jnp.float32),
        pltpu.VMEM((_NG * _LN,), jnp.int32),
        pltpu.SemaphoreType.DMA,
    ],
)
def _gather_affine(table, selc_hbm, out, st_v, out_v, selc_v, sem):
    wid = lax.axis_index("s") * 2 + lax.axis_index("c")
    pltpu.sync_copy(selc_hbm, selc_v)
    tr0 = wid * _TROWS_PER_W

    def blk_body(b, carry):
        row0 = (tr0 + b) * _RB

        copies = [
            pltpu.async_copy(
                table.at[pl.ds(row0, _RB), pl.ds(ts * 128, n * 128)],
                st_v.at[:, pl.ds(ps * 128, n * 128)], sem)
            for (ts, ps, n) in _SPANS
        ]
        for c in copies:
            c.wait()

        for g in range(_NG):
            cc = selc_v[pl.ds(g * _LN, _LN)]
            base = _GSTART[g]
            for r in range(_RB):
                rr = jnp.full((_LN,), r, jnp.int32)
                v = plsc.load_gather(st_v, [rr, cc])
                out_v[r, pl.ds(base, _LN)] = v * 12.0 - 6.0

        pltpu.sync_copy(out_v, out.at[pl.ds(row0, _RB)])
        return carry

    lax.fori_loop(0, _TROWS_PER_W, blk_body, 0)


def kernel(input_var):
    return _gather_affine(input_var, jnp.asarray(_SELC))


# speedup vs baseline: 3.5671x; 1.3868x over previous
import functools
from itertools import combinations

import numpy as np
import jax
import jax.numpy as jnp
from jax import lax
from jax.experimental import pallas as pl
from jax.experimental.pallas import tpu as pltpu
from jax.experimental.pallas import tpu_sc as plsc

_SIZE_IN = 14
_HOTNESS = 3
_BATCH = 4096
_IN_DIM = 2 ** _SIZE_IN
_NCOL = 364
_LN = 16

_NW = 32
_RB = 8
_TROWS_PER_W = (_BATCH // _RB) // _NW
_NTILE = 64
_NG = 23


def _build_constants():
    idx = np.array(
        [sum(2 ** i for i in ones) for ones in combinations(range(_SIZE_IN), _HOTNESS)],
        dtype=np.int32,
    )
    tiles = np.unique(idx >> 7)
    assert len(tiles) == _NTILE
    pos = {int(t): p for p, t in enumerate(tiles)}

    spans = []
    s = prev = int(tiles[0])
    for t in tiles[1:]:
        t = int(t)
        if t == prev + 1:
            prev = t
        else:
            spans.append((s, pos[s], prev - s + 1))
            s = prev = t
    spans.append((s, pos[s], prev - s + 1))

    gstart = [g * _LN for g in range(_NCOL // _LN)] + [_NCOL - _LN]
    selc = np.zeros(_NG * _LN, np.int32)
    for g, st in enumerate(gstart):
        for t in range(_LN):
            v = int(idx[st + t])
            selc[g * _LN + t] = pos[v >> 7] * 128 + (v & 127)
    return gstart, spans, selc


_GSTART, _SPANS, _SELC = _build_constants()

_mesh = plsc.VectorSubcoreMesh(core_axis_name="c", subcore_axis_name="s")


@functools.partial(
    pl.kernel,
    out_type=jax.ShapeDtypeStruct((_BATCH, _NCOL), jnp.float32),
    mesh=_mesh,
    compiler_params=pltpu.CompilerParams(needs_layout_passes=False,
                                         use_tc_tiling_on_sc=True),
    scratch_types=[
        pltpu.VMEM((_RB // 2, _NTILE * 128), jnp.float32),
        pltpu.VMEM((_RB // 2, _NTILE * 128), jnp.float32),
        pltpu.VMEM((_RB, _NCOL), jnp.float32),
        pltpu.VMEM((_NG * _LN,), jnp.int32),
        pltpu.SemaphoreType.DMA,
        pltpu.SemaphoreType.DMA,
    ],
)
def _gather_affine(table, selc_hbm, out, st_a, st_b, out_v, selc_v,
                   sem_a, sem_b):
    wid = lax.axis_index("s") * 2 + lax.axis_index("c")
    pltpu.sync_copy(selc_hbm, selc_v)
    tr0 = wid * _TROWS_PER_W
    hb = _RB // 2

    def issue(row0, st, sem):
        for (ts, ps, n) in _SPANS:
            pltpu.async_copy(
                table.at[pl.ds(row0, hb), pl.ds(ts * 128, n * 128)],
                st.at[:, pl.ds(ps * 128, n * 128)], sem)

    def drain(st, sem):
        pltpu.make_async_copy(
            table.at[pl.ds(0, hb), pl.ds(0, _NTILE * 128)], st, sem).wait()

    def select(st, rbase):
        for g in range(_NG):
            cc = selc_v[pl.ds(g * _LN, _LN)]
            base = _GSTART[g]
            for r in range(hb):
                rr = jnp.full((_LN,), r, jnp.int32)
                v = plsc.load_gather(st, [rr, cc])
                out_v[rbase + r, pl.ds(base, _LN)] = v * 12.0 - 6.0

    issue(tr0 * _RB, st_a, sem_a)

    def blk_body(b, carry):
        row0 = (tr0 + b) * _RB
        issue(row0 + hb, st_b, sem_b)
        drain(st_a, sem_a)
        select(st_a, 0)
        nxt = tr0 * _RB + lax.rem(row0 + _RB - tr0 * _RB,
                                  _TROWS_PER_W * _RB)
        issue(nxt, st_a, sem_a)
        drain(st_b, sem_b)
        select(st_b, hb)
        pltpu.sync_copy(out_v, out.at[pl.ds(row0, _RB)])
        return carry

    lax.fori_loop(0, _TROWS_PER_W, blk_body, 0)
    drain(st_a, sem_a)


def kernel(input_var):
    return _gather_affine(input_var, jnp.asarray(_SELC))
